# Initial kernel scaffold; baseline (speedup 1.0000x reference)
import jax
import jax.numpy as jnp
from jax.experimental import pallas as pl


def kernel(classifications, regressions, anchors, annotations):
    raise NotImplementedError("write your pallas kernel here")


# speedup vs baseline: 1.6653x; 1.6653x over previous
import functools

import jax
import jax.numpy as jnp
from jax.experimental import pallas as pl


_AB = 4464
_ALPHA = 0.5


def _focal_body(cls_ref, reg_ref, ay1_ref, ax1_ref, ay2_ref, ax2_ref,
                bx1_ref, by1_ref, bx2_ref, by2_ref, bcls_ref,
                cls_out, reg_out, npos_out):
    i = pl.program_id(1)

    ay1 = ay1_ref[...]
    ax1 = ax1_ref[...]
    ay2 = ay2_ref[...]
    ax2 = ax2_ref[...]
    bx1 = bx1_ref[0]
    by1 = by1_ref[0]
    bx2 = bx2_ref[0]
    by2 = by2_ref[0]
    bcls = bcls_ref[0]

    area_b = (bx2 - bx1) * (by2 - by1)
    iw = jnp.minimum(ax2, bx2) - jnp.maximum(ax1, bx1)
    ih = jnp.minimum(ay2, by2) - jnp.maximum(ay1, by1)
    iw = jnp.maximum(iw, 0.0)
    ih = jnp.maximum(ih, 0.0)
    inter = iw * ih
    ua = (ay2 - ay1) * (ax2 - ax1) + area_b - inter
    ua = jnp.maximum(ua, 1e-8)
    iou = inter / ua

    iou_max = jnp.max(iou, axis=1, keepdims=True)
    m_idx = jax.lax.broadcasted_iota(jnp.int32, iou.shape, 1)
    big = jnp.int32(iou.shape[1])
    argm = jnp.min(jnp.where(iou == iou_max, m_idx, big), axis=1,
                   keepdims=True)
    sel = m_idx == argm

    positive = iou_max >= 0.5
    not_ignore = positive | (iou_max < 0.4)

    def pick(v):
        return jnp.sum(jnp.where(sel, v, 0.0), axis=1, keepdims=True)

    gx1 = pick(bx1)
    gy1 = pick(by1)
    gx2 = pick(bx2)
    gy2 = pick(by2)
    gcls = pick(bcls).astype(jnp.int32)

    p = jnp.clip(cls_ref[0], 1e-4, 1.0 - 1e-4)
    neg = (1.0 - _ALPHA) * (p * p) * (-jnp.log(1.0 - p))
    row_neg = jnp.sum(neg, axis=1, keepdims=True)
    c_idx = jax.lax.broadcasted_iota(jnp.int32, p.shape, 1)
    csel = c_idx == gcls
    p_a = jnp.sum(jnp.where(csel, p, 0.0), axis=1, keepdims=True)
    neg_a = (1.0 - _ALPHA) * (p_a * p_a) * (-jnp.log(1.0 - p_a))
    pos_a = _ALPHA * (1.0 - p_a) * (1.0 - p_a) * (-jnp.log(p_a))
    cls_contrib = (jnp.where(not_ignore, row_neg, 0.0)
                   + jnp.where(positive, pos_a - neg_a, 0.0))
    s_cls = jnp.sum(cls_contrib)

    aw = ax2 - ax1
    ah = ay2 - ay1
    acx = ax1 + 0.5 * aw
    acy = ay1 + 0.5 * ah
    gw = gx2 - gx1
    gh = gy2 - gy1
    gcx = gx1 + 0.5 * gw
    gcy = gy1 + 0.5 * gh
    gw = jnp.maximum(gw, 1.0)
    gh = jnp.maximum(gh, 1.0)
    td_x = (gcx - acx) / aw
    td_y = (gcy - acy) / ah
    td_w = jnp.log(gw / aw)
    td_h = jnp.log(gh / ah)
    treg = jnp.concatenate([td_y, td_x, td_h, td_w], axis=1)
    diff = jnp.abs(treg - reg_ref[0])
    rl = jnp.where(diff <= 1.0 / 9.0, 0.5 * 9.0 * diff * diff,
                   diff - 0.5 / 9.0)
    s_reg = jnp.sum(jnp.where(positive, rl, 0.0))
    s_np = jnp.sum(jnp.where(positive, 1.0, 0.0))

    vc = jnp.full((1, 128), s_cls, jnp.float32)
    vr = jnp.full((1, 128), s_reg, jnp.float32)
    vn = jnp.full((1, 128), s_np, jnp.float32)

    @pl.when(i == 0)
    def _():
        cls_out[0] = vc
        reg_out[0] = vr
        npos_out[0] = vn

    @pl.when(i > 0)
    def _():
        cls_out[0] = cls_out[0] + vc
        reg_out[0] = reg_out[0] + vr
        npos_out[0] = npos_out[0] + vn


@functools.partial(jax.jit, static_argnames=())
def kernel(classifications, regressions, anchors, annotations):
    B, A, C = classifications.shape
    M = annotations.shape[1]
    nA = pl.cdiv(A, _AB)

    anc = anchors[0]
    ay1 = anc[:, 0:1]
    ax1 = anc[:, 1:2]
    ay2 = anc[:, 2:3]
    ax2 = anc[:, 3:4]

    bx1 = annotations[:, None, :, 0]
    by1 = annotations[:, None, :, 1]
    bx2 = annotations[:, None, :, 2]
    by2 = annotations[:, None, :, 3]
    bcls = annotations[:, None, :, 4]

    anc_spec = pl.BlockSpec((_AB, 1), lambda j, i: (i, 0))
    box_spec = pl.BlockSpec((1, 1, M), lambda j, i: (j, 0, 0))
    out_spec = pl.BlockSpec((1, 1, 128), lambda j, i: (j, 0, 0))
    out_sd = jax.ShapeDtypeStruct((B, 1, 128), jnp.float32)

    s_cls, s_reg, s_np = pl.pallas_call(
        _focal_body,
        grid=(B, nA),
        in_specs=[
            pl.BlockSpec((1, _AB, C), lambda j, i: (j, i, 0)),
            pl.BlockSpec((1, _AB, 4), lambda j, i: (j, i, 0)),
            anc_spec, anc_spec, anc_spec, anc_spec,
            box_spec, box_spec, box_spec, box_spec, box_spec,
        ],
        out_specs=[out_spec, out_spec, out_spec],
        out_shape=[out_sd, out_sd, out_sd],
    )(classifications, regressions, ay1, ax1, ay2, ax2,
      bx1, by1, bx2, by2, bcls)

    s_cls = s_cls[:, 0, 0]
    s_reg = s_reg[:, 0, 0]
    s_np = s_np[:, 0, 0]
    cls_out = jnp.mean(s_cls / jnp.maximum(s_np, 1.0), keepdims=True)
    reg_out = jnp.mean(s_reg / jnp.maximum(s_np * 4.0, 1.0), keepdims=True)
    return cls_out, reg_out


# speedup vs baseline: 7.1268x; 4.2797x over previous
import jax
import jax.numpy as jnp
from jax.experimental import pallas as pl
from jax.experimental.pallas import tpu as pltpu

_SUB = 8
_ABL = 1536


def _match_body(ay1_ref, ax1_ref, ay2_ref, ax2_ref,
                r0_ref, r1_ref, r2_ref, r3_ref, ann_ref,
                mask_ref, pos_ref, gcls_ref, reg_ref, np_ref,
                *, num_anchors, num_boxes):
    ay1 = ay1_ref[0]
    ax1 = ax1_ref[0]
    ay2 = ay2_ref[0]
    ax2 = ax2_ref[0]
    area_a = (ay2 - ay1) * (ax2 - ax1)

    best = jnp.full(ay1.shape, -1.0, jnp.float32)
    gx1 = jnp.zeros(ay1.shape, jnp.float32)
    gy1 = jnp.zeros(ay1.shape, jnp.float32)
    gx2 = jnp.zeros(ay1.shape, jnp.float32)
    gy2 = jnp.zeros(ay1.shape, jnp.float32)
    gcl = jnp.zeros(ay1.shape, jnp.float32)
    for m in range(num_boxes):
        sx1 = ann_ref[0, m, 0]
        sy1 = ann_ref[0, m, 1]
        sx2 = ann_ref[0, m, 2]
        sy2 = ann_ref[0, m, 3]
        scl = ann_ref[0, m, 4]
        iw = jnp.minimum(ax2, sx2) - jnp.maximum(ax1, sx1)
        ih = jnp.minimum(ay2, sy2) - jnp.maximum(ay1, sy1)
        iw = jnp.maximum(iw, 0.0)
        ih = jnp.maximum(ih, 0.0)
        inter = iw * ih
        ua = jnp.maximum(area_a + (sx2 - sx1) * (sy2 - sy1) - inter, 1e-8)
        iou = inter / ua
        upd = iou > best
        best = jnp.where(upd, iou, best)
        gx1 = jnp.where(upd, sx1, gx1)
        gy1 = jnp.where(upd, sy1, gy1)
        gx2 = jnp.where(upd, sx2, gx2)
        gy2 = jnp.where(upd, sy2, gy2)
        gcl = jnp.where(upd, scl, gcl)

    cols = ay1.shape[1]
    aidx = (jax.lax.broadcasted_iota(jnp.int32, ay1.shape, 0) * cols
            + jax.lax.broadcasted_iota(jnp.int32, ay1.shape, 1))
    valid = aidx < num_anchors
    posb = (best >= 0.5) & valid
    maskb = (posb | (best < 0.4)) & valid

    mask_ref[0] = jnp.where(maskb, 1.0, 0.0)
    posf = jnp.where(posb, 1.0, 0.0)
    pos_ref[0] = posf
    gcls_ref[0] = gcl

    aw = ax2 - ax1
    ah = ay2 - ay1
    acx = ax1 + 0.5 * aw
    acy = ay1 + 0.5 * ah
    gw = gx2 - gx1
    gh = gy2 - gy1
    gcx = gx1 + 0.5 * gw
    gcy = gy1 + 0.5 * gh
    gw = jnp.maximum(gw, 1.0)
    gh = jnp.maximum(gh, 1.0)
    td_y = (gcy - acy) / ah
    td_x = (gcx - acx) / aw
    td_h = jnp.log(gh / ah)
    td_w = jnp.log(gw / aw)
    rl = 0.0
    for td, r_ref in ((td_y, r0_ref), (td_x, r1_ref),
                      (td_h, r2_ref), (td_w, r3_ref)):
        d = jnp.abs(td - r_ref[0])
        rl = rl + jnp.where(d <= 1.0 / 9.0, 4.5 * d * d, d - 0.5 / 9.0)
    s_reg = jnp.sum(jnp.where(posb, rl, 0.0))
    reg_ref[0] = jnp.full((1, 128), s_reg, jnp.float32)
    np_ref[0] = jnp.full((1, 128), jnp.sum(posf), jnp.float32)


def _focal_body(cls_ref, mask_ref, pos_ref, gcls_ref, out_ref):
    i = pl.program_id(1)
    p = jnp.clip(cls_ref[0], 1e-4, 1.0 - 1e-4)
    q = 1.0 - p
    lq = jnp.log(q)
    negv = (p * p) * lq
    row_neg = jnp.sum(negv, axis=0, keepdims=True)
    s_main = -0.5 * jnp.sum(mask_ref[0] * row_neg)

    c_iota = jax.lax.broadcasted_iota(jnp.int32, p.shape, 0)
    csel = c_iota == gcls_ref[0].astype(jnp.int32)
    p_a = jnp.sum(jnp.where(csel, p, 0.0), axis=0, keepdims=True)
    q_a = 1.0 - p_a
    corr = pos_ref[0] * (0.5 * (q_a * q_a) * (-jnp.log(p_a))
                         - 0.5 * (p_a * p_a) * (-jnp.log(q_a)))
    s_blk = s_main + jnp.sum(corr)
    vc = jnp.full((1, 128), s_blk, jnp.float32)

    @pl.when(i == 0)
    def _():
        out_ref[0] = vc

    @pl.when(i > 0)
    def _():
        out_ref[0] = out_ref[0] + vc


def kernel(classifications, regressions, anchors, annotations):
    import functools

    B, A, C = classifications.shape
    M = annotations.shape[1]
    Ap = ((A + _ABL - 1) // _ABL) * _ABL
    padn = Ap - A
    cols = Ap // _SUB

    def lanes(x, pad_width):
        x = jnp.pad(x, tuple((0, 0) for _ in x.shape[:-1]) + ((0, pad_width),))
        return x.reshape(x.shape[:-1] + (_SUB, cols))

    anc = anchors[0]
    ay1 = lanes(anc[:, 0][None], padn)
    ax1 = lanes(anc[:, 1][None], padn)
    ay2 = lanes(anc[:, 2][None], padn)
    ax2 = lanes(anc[:, 3][None], padn)
    regs = [lanes(regressions[:, :, k], padn) for k in range(4)]

    anc_spec = pl.BlockSpec((1, _SUB, cols), lambda j: (0, 0, 0))
    reg_spec = pl.BlockSpec((1, _SUB, cols), lambda j: (j, 0, 0))
    ann_spec = pl.BlockSpec((1, M, 5), lambda j: (j, 0, 0),
                            memory_space=pltpu.SMEM)
    lane_out = pl.BlockSpec((1, _SUB, cols), lambda j: (j, 0, 0))
    acc_spec1 = pl.BlockSpec((1, 1, 128), lambda j: (j, 0, 0))
    lane_sd = jax.ShapeDtypeStruct((B, _SUB, cols), jnp.float32)
    acc_sd = jax.ShapeDtypeStruct((B, 1, 128), jnp.float32)

    maskf, posf, gclsf, s_reg, s_np = pl.pallas_call(
        functools.partial(_match_body, num_anchors=A, num_boxes=M),
        grid=(B,),
        in_specs=[anc_spec] * 4 + [reg_spec] * 4 + [ann_spec],
        out_specs=[lane_out, lane_out, lane_out, acc_spec1, acc_spec1],
        out_shape=[lane_sd, lane_sd, lane_sd, acc_sd, acc_sd],
    )(ay1, ax1, ay2, ax2, *regs, annotations)

    clsT = jnp.pad(jnp.transpose(classifications, (0, 2, 1)),
                   ((0, 0), (0, 0), (0, padn)))
    nABL = Ap // _ABL
    mask2 = maskf.reshape(B, 1, Ap)
    pos2 = posf.reshape(B, 1, Ap)
    gcls2 = gclsf.reshape(B, 1, Ap)

    per_anchor = pl.BlockSpec((1, 1, _ABL), lambda j, i: (j, 0, i))
    c_sum = pl.pallas_call(
        _focal_body,
        grid=(B, nABL),
        in_specs=[
            pl.BlockSpec((1, C, _ABL), lambda j, i: (j, 0, i)),
            per_anchor, per_anchor, per_anchor,
        ],
        out_specs=pl.BlockSpec((1, 1, 128), lambda j, i: (j, 0, 0)),
        out_shape=jax.ShapeDtypeStruct((B, 1, 128), jnp.float32),
    )(clsT, mask2, pos2, gcls2)

    npos = s_np[:, 0, 0]
    cls_out = jnp.mean(c_sum[:, 0, 0] / jnp.maximum(npos, 1.0), keepdims=True)
    reg_out = jnp.mean(s_reg[:, 0, 0] / jnp.maximum(npos * 4.0, 1.0),
                       keepdims=True)
    return cls_out, reg_out
